# baseline (speedup 1.0000x reference)
import functools

import jax
import jax.numpy as jnp
from jax import lax
from jax.experimental import pallas as pl
from jax.experimental.pallas import tpu as pltpu
from jax.experimental.pallas import tpu_sc as plsc

N = 100000
E = 1600000
SEQ = 8
IN = 16
H = 32

NC = 2
NS = 16
NHALF = N // NC
ACC_ROWS = 50056
TRASH = 50048
TPT = 3128
TPT_LO = 3080

SUB = 128
E_PAD = 1605632
EROWS = E_PAD // SUB

CR_CONV = 4
CPT_CONV = 196
CR_DEG = 16
CPT_DEG = 49
CR_MLP = 8
CPT_MLP = 49

BN = 2000
BE = 8192


def _sc_mesh():
    return plsc.VectorSubcoreMesh(core_axis_name="c", subcore_axis_name="s")


def _localize(idx_ref, b, nrows, coff):
    def body(j, _):
        r = j >> 3
        q = (j & 7) * 16
        d = idx_ref[b, r, pl.ds(q, 16)]
        dl = d - coff
        ok = (dl >= 0) & (dl < NHALF)
        idx_ref[b, r, pl.ds(q, 16)] = jnp.where(ok, dl, TRASH)
        return 0

    lax.fori_loop(0, nrows * 8, body, 0)


@functools.partial(
    pl.kernel,
    out_type=jax.ShapeDtypeStruct((N,), jnp.float32),
    mesh=_sc_mesh(),
    compiler_params=pltpu.CompilerParams(use_tc_tiling_on_sc=False),
    scratch_types=[
        pltpu.VMEM((2, CR_DEG, SUB), jnp.int32),
        pltpu.VMEM((TPT,), jnp.float32),
        pltpu.VMEM_SHARED((ACC_ROWS,), jnp.float32),
        pltpu.SemaphoreType.DMA,
        pltpu.SemaphoreType.DMA,
    ],
)
def _deg_sc(dst2, ones_hbm, deg_out, didx, vstage, acc, ssem, isem):
    c = lax.axis_index("c")
    s = lax.axis_index("s")
    coff = c * NHALF
    pltpu.sync_copy(ones_hbm.at[pl.ds(0, TPT)], vstage)
    pltpu.sync_copy(vstage, acc.at[pl.ds(s * TPT, TPT)])
    plsc.subcore_barrier()
    ones_v = vstage.at[pl.ds(0, SUB)]

    def _ifetch(i, b):
        rowbase = (s * CPT_DEG + i) * CR_DEG
        pltpu.async_copy(dst2.at[pl.ds(rowbase, CR_DEG)], didx.at[b], isem)

    def _iwait(i, b):
        rowbase = (s * CPT_DEG + i) * CR_DEG
        pltpu.make_async_copy(dst2.at[pl.ds(rowbase, CR_DEG)],
                              didx.at[b], isem).wait()

    _ifetch(0, 0)

    def chunk(i, _):
        b = i & 1

        @pl.when(i > 0)
        def _():
            for k in range(CR_DEG):
                pltpu.make_async_copy(ones_v, acc.at[didx.at[1 - b, k]],
                                      ssem).wait()

        _ifetch(jnp.minimum(i + 1, CPT_DEG - 1), 1 - b)
        _iwait(i, b)
        _localize(didx, b, CR_DEG, coff)
        for k in range(CR_DEG):
            pltpu.async_copy(ones_v, acc.at[didx.at[b, k]], ssem, add=True)
        return 0

    lax.fori_loop(0, CPT_DEG, chunk, 0)
    bl = (CPT_DEG - 1) & 1
    for k in range(CR_DEG):
        pltpu.make_async_copy(ones_v, acc.at[didx.at[bl, k]], ssem).wait()
    _iwait(CPT_DEG - 1, 1 - bl)
    plsc.subcore_barrier()
    base = c * NHALF + s * TPT
    pltpu.sync_copy(acc.at[pl.ds(s * TPT, TPT_LO)], vstage.at[pl.ds(0, TPT_LO)])
    pltpu.sync_copy(vstage.at[pl.ds(0, TPT_LO)], deg_out.at[pl.ds(base, TPT_LO)])

    @pl.when(s < NS - 1)
    def _():
        pltpu.sync_copy(acc.at[pl.ds(s * TPT + TPT_LO, TPT - TPT_LO)],
                        vstage.at[pl.ds(0, TPT - TPT_LO)])
        pltpu.sync_copy(vstage.at[pl.ds(0, TPT - TPT_LO)],
                        deg_out.at[pl.ds(base + TPT_LO, TPT - TPT_LO)])


@functools.partial(
    pl.kernel,
    out_type=jax.ShapeDtypeStruct((N, H), jnp.float32),
    mesh=_sc_mesh(),
    compiler_params=pltpu.CompilerParams(use_tc_tiling_on_sc=False),
    scratch_types=[
        pltpu.VMEM((2, CR_CONV, SUB), jnp.int32),
        pltpu.VMEM((2, CR_CONV, SUB), jnp.int32),
        pltpu.VMEM((CR_CONV * SUB, H), jnp.float32),
        pltpu.VMEM_SHARED((ACC_ROWS, H), jnp.float32),
        pltpu.SemaphoreType.DMA,
        pltpu.SemaphoreType.DMA,
        pltpu.SemaphoreType.DMA,
    ],
)
def _conv_sc(g, src2, dst2, s_out, sidx, didx, rows, acc, gsem, ssem, isem):
    c = lax.axis_index("c")
    s = lax.axis_index("s")
    coff = c * NHALF

    def _stage(src_ref, src_base, dst_ref, dst_base):
        piece0 = CR_CONV * SUB
        off = 0
        for piece in (piece0,) * 6 + (TPT_LO - 6 * piece0,):
            pltpu.sync_copy(src_ref.at[pl.ds(src_base + off, piece)],
                            rows.at[pl.ds(0, piece)])
            pltpu.sync_copy(rows.at[pl.ds(0, piece)],
                            dst_ref.at[pl.ds(dst_base + off, piece)])
            off += piece

        @pl.when(s < NS - 1)
        def _():
            pltpu.sync_copy(src_ref.at[pl.ds(src_base + TPT_LO, TPT - TPT_LO)],
                            rows.at[pl.ds(0, TPT - TPT_LO)])
            pltpu.sync_copy(rows.at[pl.ds(0, TPT - TPT_LO)],
                            dst_ref.at[pl.ds(dst_base + TPT_LO, TPT - TPT_LO)])

    _stage(g, coff + s * TPT, acc, s * TPT)
    plsc.subcore_barrier()

    def _ifetch(i, b):
        rowbase = (s * CPT_CONV + i) * CR_CONV
        pltpu.async_copy(src2.at[pl.ds(rowbase, CR_CONV)], sidx.at[b], isem)
        pltpu.async_copy(dst2.at[pl.ds(rowbase, CR_CONV)], didx.at[b], isem)

    def _iwait(i, b):
        rowbase = (s * CPT_CONV + i) * CR_CONV
        pltpu.make_async_copy(src2.at[pl.ds(rowbase, CR_CONV)],
                              sidx.at[b], isem).wait()
        pltpu.make_async_copy(dst2.at[pl.ds(rowbase, CR_CONV)],
                              didx.at[b], isem).wait()

    _ifetch(0, 0)

    def chunk(i, _):
        b = i & 1

        @pl.when(i > 0)
        def _():
            for k in range(CR_CONV):
                pltpu.make_async_copy(rows.at[pl.ds(k * SUB, SUB)],
                                      acc.at[didx.at[1 - b, k]], ssem).wait()

        _ifetch(jnp.minimum(i + 1, CPT_CONV - 1), 1 - b)
        _iwait(i, b)
        _localize(didx, b, CR_CONV, coff)
        pltpu.async_copy(g.at[sidx.at[b, 0]], rows.at[pl.ds(0, SUB)], gsem)
        for k in range(CR_CONV):
            if k + 1 < CR_CONV:
                pltpu.async_copy(g.at[sidx.at[b, k + 1]],
                                 rows.at[pl.ds((k + 1) * SUB, SUB)], gsem)
            pltpu.make_async_copy(g.at[sidx.at[b, k]],
                                  rows.at[pl.ds(k * SUB, SUB)], gsem).wait()
            pltpu.async_copy(rows.at[pl.ds(k * SUB, SUB)],
                             acc.at[didx.at[b, k]], ssem, add=True)
        return 0

    lax.fori_loop(0, CPT_CONV, chunk, 0)
    bl = (CPT_CONV - 1) & 1
    for k in range(CR_CONV):
        pltpu.make_async_copy(rows.at[pl.ds(k * SUB, SUB)],
                              acc.at[didx.at[bl, k]], ssem).wait()
    _iwait(CPT_CONV - 1, 1 - bl)
    plsc.subcore_barrier()
    _stage(acc, s * TPT, s_out, coff + s * TPT)


@functools.partial(
    pl.kernel,
    out_type=jax.ShapeDtypeStruct((E_PAD, H), jnp.float32),
    mesh=_sc_mesh(),
    compiler_params=pltpu.CompilerParams(use_tc_tiling_on_sc=False),
    scratch_types=[
        pltpu.VMEM((2, CR_MLP, SUB), jnp.int32),
        pltpu.VMEM((2, CR_MLP, SUB), jnp.int32),
        pltpu.VMEM((CR_MLP * SUB, H), jnp.float32),
        pltpu.SemaphoreType.DMA,
        pltpu.SemaphoreType.DMA,
        pltpu.SemaphoreType.DMA,
        pltpu.SemaphoreType.DMA,
    ],
)
def _mlp_sc(a, cc, src2, dst2, u_out, sidx, didx, rows, asem, csem, osem, isem):
    c = lax.axis_index("c")
    s = lax.axis_index("s")
    wid = c * NS + s

    def _ifetch(i, b):
        rowbase = (wid * CPT_MLP + i) * CR_MLP
        pltpu.async_copy(src2.at[pl.ds(rowbase, CR_MLP)], sidx.at[b], isem)
        pltpu.async_copy(dst2.at[pl.ds(rowbase, CR_MLP)], didx.at[b], isem)

    def _iwait(i, b):
        rowbase = (wid * CPT_MLP + i) * CR_MLP
        pltpu.make_async_copy(src2.at[pl.ds(rowbase, CR_MLP)],
                              sidx.at[b], isem).wait()
        pltpu.make_async_copy(dst2.at[pl.ds(rowbase, CR_MLP)],
                              didx.at[b], isem).wait()

    def _ocopy(i):
        rowbase = (wid * CPT_MLP + i) * CR_MLP
        return (rows, u_out.at[pl.ds(rowbase * SUB, CR_MLP * SUB)], osem)

    _ifetch(0, 0)

    def chunk(i, _):
        b = i & 1

        @pl.when(i > 0)
        def _():
            so, do, _sem = _ocopy(i - 1)
            pltpu.make_async_copy(so, do, _sem).wait()

        _ifetch(jnp.minimum(i + 1, CPT_MLP - 1), 1 - b)
        _iwait(i, b)
        for k in range(CR_MLP):
            pltpu.async_copy(a.at[sidx.at[b, k]],
                             rows.at[pl.ds(k * SUB, SUB)], asem)
        for k in range(CR_MLP):
            pltpu.make_async_copy(a.at[sidx.at[b, k]],
                                  rows.at[pl.ds(k * SUB, SUB)], asem).wait()
            pltpu.async_copy(cc.at[didx.at[b, k]],
                             rows.at[pl.ds(k * SUB, SUB)], csem, add=True)
        for k in range(CR_MLP):
            pltpu.make_async_copy(cc.at[didx.at[b, k]],
                                  rows.at[pl.ds(k * SUB, SUB)], csem).wait()
        so, do, _sem = _ocopy(i)
        pltpu.async_copy(so, do, _sem)
        return 0

    lax.fori_loop(0, CPT_MLP, chunk, 0)
    so, do, _sem = _ocopy(CPT_MLP - 1)
    pltpu.make_async_copy(so, do, _sem).wait()
    _iwait(CPT_MLP - 1, CPT_MLP & 1)


def _gru_body(x_ref, deg_ref, wir, wiz, win, whr, whz, whn,
              br, bz, bin_, bhn, w1, dinv_ref, g_ref):
    blk = x_ref.shape[0]
    h = jnp.zeros((blk, H), jnp.float32)
    f32 = jnp.float32
    for t in range(SEQ):
        xt = x_ref[:, pl.ds(t * IN, IN)]
        r = jax.nn.sigmoid(jnp.dot(xt, wir[...], preferred_element_type=f32)
                           + jnp.dot(h, whr[...], preferred_element_type=f32)
                           + br[...])
        z = jax.nn.sigmoid(jnp.dot(xt, wiz[...], preferred_element_type=f32)
                           + jnp.dot(h, whz[...], preferred_element_type=f32)
                           + bz[...])
        n = jnp.tanh(jnp.dot(xt, win[...], preferred_element_type=f32) + bin_[...]
                     + r * (jnp.dot(h, whn[...], preferred_element_type=f32)
                            + bhn[...]))
        h = (1.0 - z) * n + z * h
    dinv = lax.rsqrt(deg_ref[...])
    dinv_ref[...] = dinv
    g_ref[...] = jnp.dot(h, w1[...], preferred_element_type=f32) * dinv


def _gru_tc(x2, deg, wir, wiz, win, whr, whz, whn, br, bz, bin_, bhn, w1):
    grid = N // BN
    wspec16 = pl.BlockSpec((IN, H), lambda i: (0, 0))
    wspec32 = pl.BlockSpec((H, H), lambda i: (0, 0))
    bspec = pl.BlockSpec((1, H), lambda i: (0, 0))
    return pl.pallas_call(
        _gru_body,
        grid=(grid,),
        in_specs=[pl.BlockSpec((BN, SEQ * IN), lambda i: (i, 0)),
                  pl.BlockSpec((BN, 1), lambda i: (i, 0)),
                  wspec16, wspec16, wspec16, wspec32, wspec32, wspec32,
                  bspec, bspec, bspec, bspec, wspec32],
        out_specs=[pl.BlockSpec((BN, 1), lambda i: (i, 0)),
                   pl.BlockSpec((BN, H), lambda i: (i, 0))],
        out_shape=[jax.ShapeDtypeStruct((N, 1), jnp.float32),
                   jax.ShapeDtypeStruct((N, H), jnp.float32)],
    )(x2, deg, wir, wiz, win, whr, whz, whn, br, bz, bin_, bhn, w1)


def _mid_body(s_ref, dinv_ref, b_ref, w_ref, g_ref):
    h1 = jax.nn.relu(dinv_ref[...] * s_ref[...] + b_ref[...])
    g_ref[...] = jnp.dot(h1, w_ref[...],
                         preferred_element_type=jnp.float32) * dinv_ref[...]


def _mid_tc(s1, dinv, b, w):
    grid = N // BN
    return pl.pallas_call(
        _mid_body,
        grid=(grid,),
        in_specs=[pl.BlockSpec((BN, H), lambda i: (i, 0)),
                  pl.BlockSpec((BN, 1), lambda i: (i, 0)),
                  pl.BlockSpec((1, H), lambda i: (0, 0)),
                  pl.BlockSpec((H, H), lambda i: (0, 0))],
        out_specs=pl.BlockSpec((BN, H), lambda i: (i, 0)),
        out_shape=jax.ShapeDtypeStruct((N, H), jnp.float32),
    )(s1, dinv, b, w)


def _post_body(s_ref, dinv_ref, b_ref, wa_ref, ba_ref, wc_ref, a_ref, c_ref):
    h2 = jax.nn.relu(dinv_ref[...] * s_ref[...] + b_ref[...])
    a_ref[...] = jnp.dot(h2, wa_ref[...],
                         preferred_element_type=jnp.float32) + ba_ref[...]
    c_ref[...] = jnp.dot(h2, wc_ref[...], preferred_element_type=jnp.float32)


def _post_tc(s2, dinv, b, wa, ba, wc):
    grid = N // BN
    return pl.pallas_call(
        _post_body,
        grid=(grid,),
        in_specs=[pl.BlockSpec((BN, H), lambda i: (i, 0)),
                  pl.BlockSpec((BN, 1), lambda i: (i, 0)),
                  pl.BlockSpec((1, H), lambda i: (0, 0)),
                  pl.BlockSpec((H, H), lambda i: (0, 0)),
                  pl.BlockSpec((1, H), lambda i: (0, 0)),
                  pl.BlockSpec((H, H), lambda i: (0, 0))],
        out_specs=[pl.BlockSpec((BN, H), lambda i: (i, 0)),
                   pl.BlockSpec((BN, H), lambda i: (i, 0))],
        out_shape=[jax.ShapeDtypeStruct((N, H), jnp.float32),
                   jax.ShapeDtypeStruct((N, H), jnp.float32)],
    )(s2, dinv, b, wa, ba, wc)


def _final_body(u_ref, w_ref, b_ref, out_ref):
    out_ref[...] = (jnp.dot(jax.nn.relu(u_ref[...]), w_ref[...],
                            preferred_element_type=jnp.float32) + b_ref[...])


def _final_tc(u, w2, b2):
    grid = E_PAD // BE
    return pl.pallas_call(
        _final_body,
        grid=(grid,),
        in_specs=[pl.BlockSpec((BE, H), lambda i: (i, 0)),
                  pl.BlockSpec((H, 1), lambda i: (0, 0)),
                  pl.BlockSpec((1, 1), lambda i: (0, 0))],
        out_specs=pl.BlockSpec((BE, 1), lambda i: (i, 0)),
        out_shape=jax.ShapeDtypeStruct((E_PAD, 1), jnp.float32),
    )(u, w2, b2)


def kernel(x, edge_index, gru_W_ih, gru_W_hh, gru_b_ih, gru_b_hh,
           conv1_W, conv1_b, conv2_W, conv2_b,
           mlp_W1, mlp_b1, mlp_W2, mlp_b2):
    f32 = jnp.float32
    x2 = x.reshape(N, SEQ * IN)

    wir = gru_W_ih[:H].T
    wiz = gru_W_ih[H:2 * H].T
    win = gru_W_ih[2 * H:].T
    whr = gru_W_hh[:H].T
    whz = gru_W_hh[H:2 * H].T
    whn = gru_W_hh[2 * H:].T
    br = (gru_b_ih[:H] + gru_b_hh[:H]).reshape(1, H)
    bz = (gru_b_ih[H:2 * H] + gru_b_hh[H:2 * H]).reshape(1, H)
    bin_ = gru_b_ih[2 * H:].reshape(1, H)
    bhn = gru_b_hh[2 * H:].reshape(1, H)

    src = edge_index[0]
    dst = edge_index[1]
    pad = E_PAD - E
    src_p = jnp.concatenate([src, jnp.zeros((pad,), jnp.int32)]).reshape(EROWS, SUB)
    dst_conv = jnp.concatenate([dst, jnp.full((pad,), N, jnp.int32)]).reshape(EROWS, SUB)
    dst_mlp = jnp.concatenate([dst, jnp.zeros((pad,), jnp.int32)]).reshape(EROWS, SUB)
    ones_hbm = jnp.ones((TPT,), f32)

    deg = _deg_sc(dst_conv, ones_hbm)
    dinv, g1 = _gru_tc(x2, deg.reshape(N, 1), wir, wiz, win, whr, whz, whn,
                       br, bz, bin_, bhn, conv1_W)
    s1 = _conv_sc(g1, src_p, dst_conv)
    g2 = _mid_tc(s1, dinv, conv1_b.reshape(1, H), conv2_W)
    s2 = _conv_sc(g2, src_p, dst_conv)
    a, cc = _post_tc(s2, dinv, conv2_b.reshape(1, H),
                     mlp_W1[:H], mlp_b1.reshape(1, H), mlp_W1[H:])
    u = _mlp_sc(a, cc, src_p, dst_mlp)
    logits = _final_tc(u, mlp_W2, mlp_b2.reshape(1, 1))
    return logits[:E, 0]


# speedup vs baseline: 1.5013x; 1.5013x over previous
import functools

import jax
import jax.numpy as jnp
from jax import lax
from jax.experimental import pallas as pl
from jax.experimental.pallas import tpu as pltpu
from jax.experimental.pallas import tpu_sc as plsc

N = 100000
E = 1600000
SEQ = 8
IN = 16
H = 32

NC = 2
NS = 16
NHALF = N // NC
ACC_ROWS = 50176
TRASH = 50048
TPT = 3128
TPT_LO = 3080

SUB = 128
E_PAD = 1605632
EROWS = E_PAD // SUB

CR_CONV = 4
CPT_CONV = 196
CR_DEG = 16
CPT_DEG = 49
CR_MLP = 8
CPT_MLP = 49

BN = 2000
BE = 8192


def _sc_mesh():
    return plsc.VectorSubcoreMesh(core_axis_name="c", subcore_axis_name="s")


def _localize(idx_ref, b, nrows, coff):
    trash = TRASH + lax.iota(jnp.int32, 16)

    def body(j, _):
        r = j >> 3
        q = (j & 7) * 16
        d = idx_ref[b, r, pl.ds(q, 16)]
        dl = d - coff
        ok = (dl >= 0) & (dl < NHALF)
        idx_ref[b, r, pl.ds(q, 16)] = jnp.where(ok, dl, trash + (j & 7) * 16)
        return 0

    lax.fori_loop(0, nrows * 8, body, 0)


@functools.partial(
    pl.kernel,
    out_type=jax.ShapeDtypeStruct((N,), jnp.float32),
    mesh=_sc_mesh(),
    compiler_params=pltpu.CompilerParams(use_tc_tiling_on_sc=False),
    scratch_types=[
        pltpu.VMEM((2, CR_DEG, SUB), jnp.int32),
        pltpu.VMEM((TPT,), jnp.float32),
        pltpu.VMEM_SHARED((ACC_ROWS,), jnp.float32),
        pltpu.SemaphoreType.DMA,
        pltpu.SemaphoreType.DMA,
    ],
)
def _deg_sc(dst2, ones_hbm, deg_out, didx, vstage, acc, ssem, isem):
    c = lax.axis_index("c")
    s = lax.axis_index("s")
    coff = c * NHALF
    pltpu.sync_copy(ones_hbm.at[pl.ds(0, TPT)], vstage)
    pltpu.sync_copy(vstage, acc.at[pl.ds(s * TPT, TPT)])
    plsc.subcore_barrier()
    ones_v = vstage.at[pl.ds(0, SUB)]

    def _ifetch(i, b):
        rowbase = (s * CPT_DEG + i) * CR_DEG
        pltpu.async_copy(dst2.at[pl.ds(rowbase, CR_DEG)], didx.at[b], isem)

    def _iwait(i, b):
        rowbase = (s * CPT_DEG + i) * CR_DEG
        pltpu.make_async_copy(dst2.at[pl.ds(rowbase, CR_DEG)],
                              didx.at[b], isem).wait()

    _ifetch(0, 0)

    def chunk(i, _):
        b = i & 1

        @pl.when(i > 0)
        def _():
            for k in range(CR_DEG):
                pltpu.make_async_copy(ones_v, acc.at[didx.at[1 - b, k]],
                                      ssem).wait()

        _ifetch(jnp.minimum(i + 1, CPT_DEG - 1), 1 - b)
        _iwait(i, b)
        _localize(didx, b, CR_DEG, coff)
        for k in range(CR_DEG):
            pltpu.async_copy(ones_v, acc.at[didx.at[b, k]], ssem, add=True)
        return 0

    lax.fori_loop(0, CPT_DEG, chunk, 0)
    bl = (CPT_DEG - 1) & 1
    for k in range(CR_DEG):
        pltpu.make_async_copy(ones_v, acc.at[didx.at[bl, k]], ssem).wait()
    _iwait(CPT_DEG - 1, 1 - bl)
    plsc.subcore_barrier()
    base = c * NHALF + s * TPT
    pltpu.sync_copy(acc.at[pl.ds(s * TPT, TPT_LO)], vstage.at[pl.ds(0, TPT_LO)])
    pltpu.sync_copy(vstage.at[pl.ds(0, TPT_LO)], deg_out.at[pl.ds(base, TPT_LO)])

    @pl.when(s < NS - 1)
    def _():
        pltpu.sync_copy(acc.at[pl.ds(s * TPT + TPT_LO, TPT - TPT_LO)],
                        vstage.at[pl.ds(0, TPT - TPT_LO)])
        pltpu.sync_copy(vstage.at[pl.ds(0, TPT - TPT_LO)],
                        deg_out.at[pl.ds(base + TPT_LO, TPT - TPT_LO)])


@functools.partial(
    pl.kernel,
    out_type=jax.ShapeDtypeStruct((N, H), jnp.float32),
    mesh=_sc_mesh(),
    compiler_params=pltpu.CompilerParams(use_tc_tiling_on_sc=False),
    scratch_types=[
        pltpu.VMEM((2, CR_CONV, SUB), jnp.int32),
        pltpu.VMEM((2, CR_CONV, SUB), jnp.int32),
        pltpu.VMEM((CR_CONV * SUB, H), jnp.float32),
        pltpu.VMEM_SHARED((ACC_ROWS, H), jnp.float32),
        pltpu.SemaphoreType.DMA,
        pltpu.SemaphoreType.DMA,
        pltpu.SemaphoreType.DMA,
    ],
)
def _conv_sc(g, src2, dst2, s_out, sidx, didx, rows, acc, gsem, ssem, isem):
    c = lax.axis_index("c")
    s = lax.axis_index("s")
    coff = c * NHALF

    def _stage(src_ref, src_base, dst_ref, dst_base):
        piece0 = CR_CONV * SUB
        off = 0
        for piece in (piece0,) * 6 + (TPT_LO - 6 * piece0,):
            pltpu.sync_copy(src_ref.at[pl.ds(src_base + off, piece)],
                            rows.at[pl.ds(0, piece)])
            pltpu.sync_copy(rows.at[pl.ds(0, piece)],
                            dst_ref.at[pl.ds(dst_base + off, piece)])
            off += piece

        @pl.when(s < NS - 1)
        def _():
            pltpu.sync_copy(src_ref.at[pl.ds(src_base + TPT_LO, TPT - TPT_LO)],
                            rows.at[pl.ds(0, TPT - TPT_LO)])
            pltpu.sync_copy(rows.at[pl.ds(0, TPT - TPT_LO)],
                            dst_ref.at[pl.ds(dst_base + TPT_LO, TPT - TPT_LO)])

    _stage(g, coff + s * TPT, acc, s * TPT)
    plsc.subcore_barrier()

    def _ifetch(i, b):
        rowbase = (s * CPT_CONV + i) * CR_CONV
        pltpu.async_copy(src2.at[pl.ds(rowbase, CR_CONV)], sidx.at[b], isem)
        pltpu.async_copy(dst2.at[pl.ds(rowbase, CR_CONV)], didx.at[b], isem)

    def _iwait(i, b):
        rowbase = (s * CPT_CONV + i) * CR_CONV
        pltpu.make_async_copy(src2.at[pl.ds(rowbase, CR_CONV)],
                              sidx.at[b], isem).wait()
        pltpu.make_async_copy(dst2.at[pl.ds(rowbase, CR_CONV)],
                              didx.at[b], isem).wait()

    _ifetch(0, 0)

    def chunk(i, _):
        b = i & 1

        @pl.when(i > 0)
        def _():
            for k in range(CR_CONV):
                pltpu.make_async_copy(rows.at[pl.ds(k * SUB, SUB)],
                                      acc.at[didx.at[1 - b, k]], ssem).wait()

        _ifetch(jnp.minimum(i + 1, CPT_CONV - 1), 1 - b)
        _iwait(i, b)
        _localize(didx, b, CR_CONV, coff)
        pltpu.async_copy(g.at[sidx.at[b, 0]], rows.at[pl.ds(0, SUB)], gsem)
        for k in range(CR_CONV):
            if k + 1 < CR_CONV:
                pltpu.async_copy(g.at[sidx.at[b, k + 1]],
                                 rows.at[pl.ds((k + 1) * SUB, SUB)], gsem)
            pltpu.make_async_copy(g.at[sidx.at[b, k]],
                                  rows.at[pl.ds(k * SUB, SUB)], gsem).wait()
            pltpu.async_copy(rows.at[pl.ds(k * SUB, SUB)],
                             acc.at[didx.at[b, k]], ssem, add=True)
        return 0

    lax.fori_loop(0, CPT_CONV, chunk, 0)
    bl = (CPT_CONV - 1) & 1
    for k in range(CR_CONV):
        pltpu.make_async_copy(rows.at[pl.ds(k * SUB, SUB)],
                              acc.at[didx.at[bl, k]], ssem).wait()
    _iwait(CPT_CONV - 1, 1 - bl)
    plsc.subcore_barrier()
    _stage(acc, s * TPT, s_out, coff + s * TPT)


@functools.partial(
    pl.kernel,
    out_type=jax.ShapeDtypeStruct((E_PAD, H), jnp.float32),
    mesh=_sc_mesh(),
    compiler_params=pltpu.CompilerParams(use_tc_tiling_on_sc=False),
    scratch_types=[
        pltpu.VMEM((2, CR_MLP, SUB), jnp.int32),
        pltpu.VMEM((2, CR_MLP, SUB), jnp.int32),
        pltpu.VMEM((CR_MLP * SUB, H), jnp.float32),
        pltpu.SemaphoreType.DMA,
        pltpu.SemaphoreType.DMA,
        pltpu.SemaphoreType.DMA,
        pltpu.SemaphoreType.DMA,
    ],
)
def _mlp_sc(a, cc, src2, dst2, u_out, sidx, didx, rows, asem, csem, osem, isem):
    c = lax.axis_index("c")
    s = lax.axis_index("s")
    wid = c * NS + s

    def _ifetch(i, b):
        rowbase = (wid * CPT_MLP + i) * CR_MLP
        pltpu.async_copy(src2.at[pl.ds(rowbase, CR_MLP)], sidx.at[b], isem)
        pltpu.async_copy(dst2.at[pl.ds(rowbase, CR_MLP)], didx.at[b], isem)

    def _iwait(i, b):
        rowbase = (wid * CPT_MLP + i) * CR_MLP
        pltpu.make_async_copy(src2.at[pl.ds(rowbase, CR_MLP)],
                              sidx.at[b], isem).wait()
        pltpu.make_async_copy(dst2.at[pl.ds(rowbase, CR_MLP)],
                              didx.at[b], isem).wait()

    def _ocopy(i):
        rowbase = (wid * CPT_MLP + i) * CR_MLP
        return (rows, u_out.at[pl.ds(rowbase * SUB, CR_MLP * SUB)], osem)

    _ifetch(0, 0)

    def chunk(i, _):
        b = i & 1

        @pl.when(i > 0)
        def _():
            so, do, _sem = _ocopy(i - 1)
            pltpu.make_async_copy(so, do, _sem).wait()

        _ifetch(jnp.minimum(i + 1, CPT_MLP - 1), 1 - b)
        _iwait(i, b)
        for k in range(CR_MLP):
            pltpu.async_copy(a.at[sidx.at[b, k]],
                             rows.at[pl.ds(k * SUB, SUB)], asem)
        for k in range(CR_MLP):
            pltpu.make_async_copy(a.at[sidx.at[b, k]],
                                  rows.at[pl.ds(k * SUB, SUB)], asem).wait()
            pltpu.async_copy(cc.at[didx.at[b, k]],
                             rows.at[pl.ds(k * SUB, SUB)], csem, add=True)
        for k in range(CR_MLP):
            pltpu.make_async_copy(cc.at[didx.at[b, k]],
                                  rows.at[pl.ds(k * SUB, SUB)], csem).wait()
        so, do, _sem = _ocopy(i)
        pltpu.async_copy(so, do, _sem)
        return 0

    lax.fori_loop(0, CPT_MLP, chunk, 0)
    so, do, _sem = _ocopy(CPT_MLP - 1)
    pltpu.make_async_copy(so, do, _sem).wait()
    _iwait(CPT_MLP - 1, CPT_MLP & 1)


def _gru_body(x_ref, deg_ref, wir, wiz, win, whr, whz, whn,
              br, bz, bin_, bhn, w1, dinv_ref, g_ref):
    blk = x_ref.shape[0]
    h = jnp.zeros((blk, H), jnp.float32)
    f32 = jnp.float32
    for t in range(SEQ):
        xt = x_ref[:, pl.ds(t * IN, IN)]
        r = jax.nn.sigmoid(jnp.dot(xt, wir[...], preferred_element_type=f32)
                           + jnp.dot(h, whr[...], preferred_element_type=f32)
                           + br[...])
        z = jax.nn.sigmoid(jnp.dot(xt, wiz[...], preferred_element_type=f32)
                           + jnp.dot(h, whz[...], preferred_element_type=f32)
                           + bz[...])
        n = jnp.tanh(jnp.dot(xt, win[...], preferred_element_type=f32) + bin_[...]
                     + r * (jnp.dot(h, whn[...], preferred_element_type=f32)
                            + bhn[...]))
        h = (1.0 - z) * n + z * h
    dinv = lax.rsqrt(deg_ref[...])
    dinv_ref[...] = dinv
    g_ref[...] = jnp.dot(h, w1[...], preferred_element_type=f32) * dinv


def _gru_tc(x2, deg, wir, wiz, win, whr, whz, whn, br, bz, bin_, bhn, w1):
    grid = N // BN
    wspec16 = pl.BlockSpec((IN, H), lambda i: (0, 0))
    wspec32 = pl.BlockSpec((H, H), lambda i: (0, 0))
    bspec = pl.BlockSpec((1, H), lambda i: (0, 0))
    return pl.pallas_call(
        _gru_body,
        grid=(grid,),
        in_specs=[pl.BlockSpec((BN, SEQ * IN), lambda i: (i, 0)),
                  pl.BlockSpec((BN, 1), lambda i: (i, 0)),
                  wspec16, wspec16, wspec16, wspec32, wspec32, wspec32,
                  bspec, bspec, bspec, bspec, wspec32],
        out_specs=[pl.BlockSpec((BN, 1), lambda i: (i, 0)),
                   pl.BlockSpec((BN, H), lambda i: (i, 0))],
        out_shape=[jax.ShapeDtypeStruct((N, 1), jnp.float32),
                   jax.ShapeDtypeStruct((N, H), jnp.float32)],
    )(x2, deg, wir, wiz, win, whr, whz, whn, br, bz, bin_, bhn, w1)


def _mid_body(s_ref, dinv_ref, b_ref, w_ref, g_ref):
    h1 = jax.nn.relu(dinv_ref[...] * s_ref[...] + b_ref[...])
    g_ref[...] = jnp.dot(h1, w_ref[...],
                         preferred_element_type=jnp.float32) * dinv_ref[...]


def _mid_tc(s1, dinv, b, w):
    grid = N // BN
    return pl.pallas_call(
        _mid_body,
        grid=(grid,),
        in_specs=[pl.BlockSpec((BN, H), lambda i: (i, 0)),
                  pl.BlockSpec((BN, 1), lambda i: (i, 0)),
                  pl.BlockSpec((1, H), lambda i: (0, 0)),
                  pl.BlockSpec((H, H), lambda i: (0, 0))],
        out_specs=pl.BlockSpec((BN, H), lambda i: (i, 0)),
        out_shape=jax.ShapeDtypeStruct((N, H), jnp.float32),
    )(s1, dinv, b, w)


def _post_body(s_ref, dinv_ref, b_ref, wa_ref, ba_ref, wc_ref, a_ref, c_ref):
    h2 = jax.nn.relu(dinv_ref[...] * s_ref[...] + b_ref[...])
    a_ref[...] = jnp.dot(h2, wa_ref[...],
                         preferred_element_type=jnp.float32) + ba_ref[...]
    c_ref[...] = jnp.dot(h2, wc_ref[...], preferred_element_type=jnp.float32)


def _post_tc(s2, dinv, b, wa, ba, wc):
    grid = N // BN
    return pl.pallas_call(
        _post_body,
        grid=(grid,),
        in_specs=[pl.BlockSpec((BN, H), lambda i: (i, 0)),
                  pl.BlockSpec((BN, 1), lambda i: (i, 0)),
                  pl.BlockSpec((1, H), lambda i: (0, 0)),
                  pl.BlockSpec((H, H), lambda i: (0, 0)),
                  pl.BlockSpec((1, H), lambda i: (0, 0)),
                  pl.BlockSpec((H, H), lambda i: (0, 0))],
        out_specs=[pl.BlockSpec((BN, H), lambda i: (i, 0)),
                   pl.BlockSpec((BN, H), lambda i: (i, 0))],
        out_shape=[jax.ShapeDtypeStruct((N, H), jnp.float32),
                   jax.ShapeDtypeStruct((N, H), jnp.float32)],
    )(s2, dinv, b, wa, ba, wc)


def _final_body(u_ref, w_ref, b_ref, out_ref):
    out_ref[...] = (jnp.dot(jax.nn.relu(u_ref[...]), w_ref[...],
                            preferred_element_type=jnp.float32) + b_ref[...])


def _final_tc(u, w2, b2):
    grid = E_PAD // BE
    return pl.pallas_call(
        _final_body,
        grid=(grid,),
        in_specs=[pl.BlockSpec((BE, H), lambda i: (i, 0)),
                  pl.BlockSpec((H, 1), lambda i: (0, 0)),
                  pl.BlockSpec((1, 1), lambda i: (0, 0))],
        out_specs=pl.BlockSpec((BE, 1), lambda i: (i, 0)),
        out_shape=jax.ShapeDtypeStruct((E_PAD, 1), jnp.float32),
    )(u, w2, b2)


def kernel(x, edge_index, gru_W_ih, gru_W_hh, gru_b_ih, gru_b_hh,
           conv1_W, conv1_b, conv2_W, conv2_b,
           mlp_W1, mlp_b1, mlp_W2, mlp_b2):
    f32 = jnp.float32
    x2 = x.reshape(N, SEQ * IN)

    wir = gru_W_ih[:H].T
    wiz = gru_W_ih[H:2 * H].T
    win = gru_W_ih[2 * H:].T
    whr = gru_W_hh[:H].T
    whz = gru_W_hh[H:2 * H].T
    whn = gru_W_hh[2 * H:].T
    br = (gru_b_ih[:H] + gru_b_hh[:H]).reshape(1, H)
    bz = (gru_b_ih[H:2 * H] + gru_b_hh[H:2 * H]).reshape(1, H)
    bin_ = gru_b_ih[2 * H:].reshape(1, H)
    bhn = gru_b_hh[2 * H:].reshape(1, H)

    src = edge_index[0]
    dst = edge_index[1]
    pad = E_PAD - E
    src_p = jnp.concatenate([src, jnp.zeros((pad,), jnp.int32)]).reshape(EROWS, SUB)
    dst_conv = jnp.concatenate([dst, jnp.full((pad,), N, jnp.int32)]).reshape(EROWS, SUB)
    dst_mlp = jnp.concatenate([dst, jnp.zeros((pad,), jnp.int32)]).reshape(EROWS, SUB)
    ones_hbm = jnp.ones((TPT,), f32)

    deg = _deg_sc(dst_conv, ones_hbm)
    dinv, g1 = _gru_tc(x2, deg.reshape(N, 1), wir, wiz, win, whr, whz, whn,
                       br, bz, bin_, bhn, conv1_W)
    s1 = _conv_sc(g1, src_p, dst_conv)
    g2 = _mid_tc(s1, dinv, conv1_b.reshape(1, H), conv2_W)
    s2 = _conv_sc(g2, src_p, dst_conv)
    a, cc = _post_tc(s2, dinv, conv2_b.reshape(1, H),
                     mlp_W1[:H], mlp_b1.reshape(1, H), mlp_W1[H:])
    u = _mlp_sc(a, cc, src_p, dst_mlp)
    logits = _final_tc(u, mlp_W2, mlp_b2.reshape(1, 1))
    return logits[:E, 0]


# speedup vs baseline: 1.9067x; 1.2700x over previous
import functools

import jax
import jax.numpy as jnp
from jax import lax
from jax.experimental import pallas as pl
from jax.experimental.pallas import tpu as pltpu
from jax.experimental.pallas import tpu_sc as plsc

N = 100000
E = 1600000
SEQ = 8
IN = 16
H = 32

NC = 2
NS = 16
NHALF = N // NC
ACC_ROWS = 50176
TRASH = 50048
TPT = 3128
TPT_LO = 3080

SUB = 128
E_PAD = 1605632
EROWS = E_PAD // SUB

CR_CONV = 4
CPT_CONV = 196
CR_DEG = 16
CPT_DEG = 49
CR_MLP = 8
CPT_MLP = 49

BN = 2000
BE = 8192


def _sc_mesh():
    return plsc.VectorSubcoreMesh(core_axis_name="c", subcore_axis_name="s")


def _localize(idx_ref, b, nrows, coff):
    trash = TRASH + lax.iota(jnp.int32, 16)

    def body(j, _):
        r = j >> 3
        q = (j & 7) * 16
        d = idx_ref[b, r, pl.ds(q, 16)]
        dl = d - coff
        ok = (dl >= 0) & (dl < NHALF)
        idx_ref[b, r, pl.ds(q, 16)] = jnp.where(ok, dl, trash + (j & 7) * 16)
        return 0

    lax.fori_loop(0, nrows * 8, body, 0)


@functools.partial(
    pl.kernel,
    out_type=jax.ShapeDtypeStruct((N,), jnp.float32),
    mesh=_sc_mesh(),
    compiler_params=pltpu.CompilerParams(use_tc_tiling_on_sc=False),
    scratch_types=[
        pltpu.VMEM((2, CR_DEG, SUB), jnp.int32),
        pltpu.VMEM((TPT,), jnp.float32),
        pltpu.VMEM_SHARED((ACC_ROWS,), jnp.float32),
        pltpu.SemaphoreType.DMA,
        pltpu.SemaphoreType.DMA,
    ],
)
def _deg_sc(dst2, ones_hbm, deg_out, didx, vstage, acc, ssem, isem):
    c = lax.axis_index("c")
    s = lax.axis_index("s")
    coff = c * NHALF
    pltpu.sync_copy(ones_hbm.at[pl.ds(0, TPT)], vstage)
    pltpu.sync_copy(vstage, acc.at[pl.ds(s * TPT, TPT)])
    plsc.subcore_barrier()
    ones_v = vstage.at[pl.ds(0, SUB)]

    def _ifetch(i, b):
        rowbase = (s * CPT_DEG + i) * CR_DEG
        pltpu.async_copy(dst2.at[pl.ds(rowbase, CR_DEG)], didx.at[b], isem)

    def _iwait(i, b):
        rowbase = (s * CPT_DEG + i) * CR_DEG
        pltpu.make_async_copy(dst2.at[pl.ds(rowbase, CR_DEG)],
                              didx.at[b], isem).wait()

    _ifetch(0, 0)

    def chunk(i, _):
        b = i & 1

        @pl.when(i > 0)
        def _():
            for k in range(CR_DEG):
                pltpu.make_async_copy(ones_v, acc.at[didx.at[1 - b, k]],
                                      ssem).wait()

        _ifetch(jnp.minimum(i + 1, CPT_DEG - 1), 1 - b)
        _iwait(i, b)
        _localize(didx, b, CR_DEG, coff)
        for k in range(CR_DEG):
            pltpu.async_copy(ones_v, acc.at[didx.at[b, k]], ssem, add=True)
        return 0

    lax.fori_loop(0, CPT_DEG, chunk, 0)
    bl = (CPT_DEG - 1) & 1
    for k in range(CR_DEG):
        pltpu.make_async_copy(ones_v, acc.at[didx.at[bl, k]], ssem).wait()
    _iwait(CPT_DEG - 1, 1 - bl)
    plsc.subcore_barrier()
    base = c * NHALF + s * TPT
    pltpu.sync_copy(acc.at[pl.ds(s * TPT, TPT_LO)], vstage.at[pl.ds(0, TPT_LO)])
    pltpu.sync_copy(vstage.at[pl.ds(0, TPT_LO)], deg_out.at[pl.ds(base, TPT_LO)])

    @pl.when(s < NS - 1)
    def _():
        pltpu.sync_copy(acc.at[pl.ds(s * TPT + TPT_LO, TPT - TPT_LO)],
                        vstage.at[pl.ds(0, TPT - TPT_LO)])
        pltpu.sync_copy(vstage.at[pl.ds(0, TPT - TPT_LO)],
                        deg_out.at[pl.ds(base + TPT_LO, TPT - TPT_LO)])


@functools.partial(
    pl.kernel,
    out_type=jax.ShapeDtypeStruct((N, H), jnp.float32),
    mesh=_sc_mesh(),
    compiler_params=pltpu.CompilerParams(use_tc_tiling_on_sc=False),
    scratch_types=[
        pltpu.VMEM((2, CR_CONV, SUB), jnp.int32),
        pltpu.VMEM((2, CR_CONV, SUB), jnp.int32),
        pltpu.VMEM((CR_CONV * SUB, H), jnp.float32),
        pltpu.VMEM_SHARED((ACC_ROWS, H), jnp.float32),
        pltpu.SemaphoreType.DMA,
        pltpu.SemaphoreType.DMA,
        pltpu.SemaphoreType.DMA,
    ],
)
def _conv_sc(g, src2, dst2, s_out, sidx, didx, rows, acc, gsem, ssem, isem):
    c = lax.axis_index("c")
    s = lax.axis_index("s")
    coff = c * NHALF

    def _stage(src_ref, src_base, dst_ref, dst_base):
        piece0 = CR_CONV * SUB
        off = 0
        for piece in (piece0,) * 6 + (TPT_LO - 6 * piece0,):
            pltpu.sync_copy(src_ref.at[pl.ds(src_base + off, piece)],
                            rows.at[pl.ds(0, piece)])
            pltpu.sync_copy(rows.at[pl.ds(0, piece)],
                            dst_ref.at[pl.ds(dst_base + off, piece)])
            off += piece

        @pl.when(s < NS - 1)
        def _():
            pltpu.sync_copy(src_ref.at[pl.ds(src_base + TPT_LO, TPT - TPT_LO)],
                            rows.at[pl.ds(0, TPT - TPT_LO)])
            pltpu.sync_copy(rows.at[pl.ds(0, TPT - TPT_LO)],
                            dst_ref.at[pl.ds(dst_base + TPT_LO, TPT - TPT_LO)])

    _stage(g, coff + s * TPT, acc, s * TPT)
    plsc.subcore_barrier()

    def _ifetch(i, b):
        rowbase = (s * CPT_CONV + i) * CR_CONV
        pltpu.async_copy(src2.at[pl.ds(rowbase, CR_CONV)], sidx.at[b], isem)
        pltpu.async_copy(dst2.at[pl.ds(rowbase, CR_CONV)], didx.at[b], isem)

    def _iwait(i, b):
        rowbase = (s * CPT_CONV + i) * CR_CONV
        pltpu.make_async_copy(src2.at[pl.ds(rowbase, CR_CONV)],
                              sidx.at[b], isem).wait()
        pltpu.make_async_copy(dst2.at[pl.ds(rowbase, CR_CONV)],
                              didx.at[b], isem).wait()

    _ifetch(0, 0)

    def chunk(i, _):
        b = i & 1

        @pl.when(i > 0)
        def _():
            for k in range(CR_CONV):
                pltpu.make_async_copy(rows.at[pl.ds(k * SUB, SUB)],
                                      acc.at[didx.at[1 - b, k]], ssem).wait()

        _ifetch(jnp.minimum(i + 1, CPT_CONV - 1), 1 - b)
        _iwait(i, b)
        _localize(didx, b, CR_CONV, coff)
        pltpu.async_copy(g.at[sidx.at[b, 0]], rows.at[pl.ds(0, SUB)], gsem)
        for k in range(CR_CONV):
            if k + 1 < CR_CONV:
                pltpu.async_copy(g.at[sidx.at[b, k + 1]],
                                 rows.at[pl.ds((k + 1) * SUB, SUB)], gsem)
            pltpu.make_async_copy(g.at[sidx.at[b, k]],
                                  rows.at[pl.ds(k * SUB, SUB)], gsem).wait()
            pltpu.async_copy(rows.at[pl.ds(k * SUB, SUB)],
                             acc.at[didx.at[b, k]], ssem, add=True)
        return 0

    lax.fori_loop(0, CPT_CONV, chunk, 0)
    bl = (CPT_CONV - 1) & 1
    for k in range(CR_CONV):
        pltpu.make_async_copy(rows.at[pl.ds(k * SUB, SUB)],
                              acc.at[didx.at[bl, k]], ssem).wait()
    _iwait(CPT_CONV - 1, 1 - bl)
    plsc.subcore_barrier()
    _stage(acc, s * TPT, s_out, coff + s * TPT)


CMLP = CR_MLP * SUB


@functools.partial(
    pl.kernel,
    out_type=jax.ShapeDtypeStruct((E_PAD, H), jnp.float32),
    mesh=_sc_mesh(),
    compiler_params=pltpu.CompilerParams(use_tc_tiling_on_sc=False),
    scratch_types=[
        pltpu.VMEM((2, CR_MLP, SUB), jnp.int32),
        pltpu.VMEM((2, CR_MLP, SUB), jnp.int32),
        pltpu.VMEM((2, CMLP, H), jnp.float32),
        pltpu.SemaphoreType.DMA,
        pltpu.SemaphoreType.DMA,
        pltpu.SemaphoreType.DMA,
        pltpu.SemaphoreType.DMA,
    ],
)
def _mlp_sc(a, cc, src2, dst2, u_out, sidx, didx, rows, asem, csem, osem, isem):
    c = lax.axis_index("c")
    s = lax.axis_index("s")
    wid = c * NS + s

    def _ifetch(i, b):
        rowbase = (wid * CPT_MLP + i) * CR_MLP
        pltpu.async_copy(src2.at[pl.ds(rowbase, CR_MLP)], sidx.at[b], isem)
        pltpu.async_copy(dst2.at[pl.ds(rowbase, CR_MLP)], didx.at[b], isem)

    def _iwait(i, b):
        rowbase = (wid * CPT_MLP + i) * CR_MLP
        pltpu.make_async_copy(src2.at[pl.ds(rowbase, CR_MLP)],
                              sidx.at[b], isem).wait()
        pltpu.make_async_copy(dst2.at[pl.ds(rowbase, CR_MLP)],
                              didx.at[b], isem).wait()

    def _ocopy(i):
        base = (wid * CPT_MLP + i) * CMLP
        return (rows.at[i & 1], u_out.at[pl.ds(base, CMLP)], osem)

    _ifetch(0, 0)

    def chunk(i, _):
        b = i & 1

        @pl.when(i > 1)
        def _():
            so, do, _sem = _ocopy(i - 2)
            pltpu.make_async_copy(so, do, _sem).wait()

        _ifetch(jnp.minimum(i + 1, CPT_MLP - 1), 1 - b)
        _iwait(i, b)
        for k in range(CR_MLP):
            pltpu.async_copy(a.at[sidx.at[b, k]],
                             rows.at[b, pl.ds(k * SUB, SUB)], asem)
        for k in range(CR_MLP):
            pltpu.make_async_copy(a.at[sidx.at[b, k]],
                                  rows.at[b, pl.ds(k * SUB, SUB)], asem).wait()
            pltpu.async_copy(cc.at[didx.at[b, k]],
                             rows.at[b, pl.ds(k * SUB, SUB)], csem, add=True)
        for k in range(CR_MLP):
            pltpu.make_async_copy(cc.at[didx.at[b, k]],
                                  rows.at[b, pl.ds(k * SUB, SUB)], csem).wait()
        so, do, _sem = _ocopy(i)
        pltpu.async_copy(so, do, _sem)
        return 0

    lax.fori_loop(0, CPT_MLP, chunk, 0)
    so, do, _sem = _ocopy(CPT_MLP - 1)
    pltpu.make_async_copy(so, do, _sem).wait()
    so2, do2, _sem2 = _ocopy(CPT_MLP - 2)
    pltpu.make_async_copy(so2, do2, _sem2).wait()
    _iwait(CPT_MLP - 1, CPT_MLP & 1)


def _gru_body(x_ref, deg_ref, wir, wiz, win, whr, whz, whn,
              br, bz, bin_, bhn, w1, dinv_ref, g_ref):
    blk = x_ref.shape[0]
    h = jnp.zeros((blk, H), jnp.float32)
    f32 = jnp.float32
    for t in range(SEQ):
        xt = x_ref[:, pl.ds(t * IN, IN)]
        r = jax.nn.sigmoid(jnp.dot(xt, wir[...], preferred_element_type=f32)
                           + jnp.dot(h, whr[...], preferred_element_type=f32)
                           + br[...])
        z = jax.nn.sigmoid(jnp.dot(xt, wiz[...], preferred_element_type=f32)
                           + jnp.dot(h, whz[...], preferred_element_type=f32)
                           + bz[...])
        n = jnp.tanh(jnp.dot(xt, win[...], preferred_element_type=f32) + bin_[...]
                     + r * (jnp.dot(h, whn[...], preferred_element_type=f32)
                            + bhn[...]))
        h = (1.0 - z) * n + z * h
    dinv = lax.rsqrt(deg_ref[...])
    dinv_ref[...] = dinv
    g_ref[...] = jnp.dot(h, w1[...], preferred_element_type=f32) * dinv


def _gru_tc(x2, deg, wir, wiz, win, whr, whz, whn, br, bz, bin_, bhn, w1):
    grid = N // BN
    wspec16 = pl.BlockSpec((IN, H), lambda i: (0, 0))
    wspec32 = pl.BlockSpec((H, H), lambda i: (0, 0))
    bspec = pl.BlockSpec((1, H), lambda i: (0, 0))
    return pl.pallas_call(
        _gru_body,
        grid=(grid,),
        in_specs=[pl.BlockSpec((BN, SEQ * IN), lambda i: (i, 0)),
                  pl.BlockSpec((BN, 1), lambda i: (i, 0)),
                  wspec16, wspec16, wspec16, wspec32, wspec32, wspec32,
                  bspec, bspec, bspec, bspec, wspec32],
        out_specs=[pl.BlockSpec((BN, 1), lambda i: (i, 0)),
                   pl.BlockSpec((BN, H), lambda i: (i, 0))],
        out_shape=[jax.ShapeDtypeStruct((N, 1), jnp.float32),
                   jax.ShapeDtypeStruct((N, H), jnp.float32)],
    )(x2, deg, wir, wiz, win, whr, whz, whn, br, bz, bin_, bhn, w1)


def _mid_body(s_ref, dinv_ref, b_ref, w_ref, g_ref):
    h1 = jax.nn.relu(dinv_ref[...] * s_ref[...] + b_ref[...])
    g_ref[...] = jnp.dot(h1, w_ref[...],
                         preferred_element_type=jnp.float32) * dinv_ref[...]


def _mid_tc(s1, dinv, b, w):
    grid = N // BN
    return pl.pallas_call(
        _mid_body,
        grid=(grid,),
        in_specs=[pl.BlockSpec((BN, H), lambda i: (i, 0)),
                  pl.BlockSpec((BN, 1), lambda i: (i, 0)),
                  pl.BlockSpec((1, H), lambda i: (0, 0)),
                  pl.BlockSpec((H, H), lambda i: (0, 0))],
        out_specs=pl.BlockSpec((BN, H), lambda i: (i, 0)),
        out_shape=jax.ShapeDtypeStruct((N, H), jnp.float32),
    )(s1, dinv, b, w)


def _post_body(s_ref, dinv_ref, b_ref, wa_ref, ba_ref, wc_ref, a_ref, c_ref):
    h2 = jax.nn.relu(dinv_ref[...] * s_ref[...] + b_ref[...])
    a_ref[...] = jnp.dot(h2, wa_ref[...],
                         preferred_element_type=jnp.float32) + ba_ref[...]
    c_ref[...] = jnp.dot(h2, wc_ref[...], preferred_element_type=jnp.float32)


def _post_tc(s2, dinv, b, wa, ba, wc):
    grid = N // BN
    return pl.pallas_call(
        _post_body,
        grid=(grid,),
        in_specs=[pl.BlockSpec((BN, H), lambda i: (i, 0)),
                  pl.BlockSpec((BN, 1), lambda i: (i, 0)),
                  pl.BlockSpec((1, H), lambda i: (0, 0)),
                  pl.BlockSpec((H, H), lambda i: (0, 0)),
                  pl.BlockSpec((1, H), lambda i: (0, 0)),
                  pl.BlockSpec((H, H), lambda i: (0, 0))],
        out_specs=[pl.BlockSpec((BN, H), lambda i: (i, 0)),
                   pl.BlockSpec((BN, H), lambda i: (i, 0))],
        out_shape=[jax.ShapeDtypeStruct((N, H), jnp.float32),
                   jax.ShapeDtypeStruct((N, H), jnp.float32)],
    )(s2, dinv, b, wa, ba, wc)


def _final_body(u_ref, w_ref, b_ref, out_ref):
    out_ref[...] = (jnp.dot(jax.nn.relu(u_ref[...]), w_ref[...],
                            preferred_element_type=jnp.float32) + b_ref[...])


def _final_tc(u4, w24, b24):
    grid = E_PAD // 4 // 2048
    return pl.pallas_call(
        _final_body,
        grid=(grid,),
        in_specs=[pl.BlockSpec((2048, 4 * H), lambda i: (i, 0)),
                  pl.BlockSpec((4 * H, 4), lambda i: (0, 0)),
                  pl.BlockSpec((1, 4), lambda i: (0, 0))],
        out_specs=pl.BlockSpec((2048, 4), lambda i: (i, 0)),
        out_shape=jax.ShapeDtypeStruct((E_PAD // 4, 4), jnp.float32),
    )(u4, w24, b24)


def kernel(x, edge_index, gru_W_ih, gru_W_hh, gru_b_ih, gru_b_hh,
           conv1_W, conv1_b, conv2_W, conv2_b,
           mlp_W1, mlp_b1, mlp_W2, mlp_b2):
    f32 = jnp.float32
    x2 = x.reshape(N, SEQ * IN)

    wir = gru_W_ih[:H].T
    wiz = gru_W_ih[H:2 * H].T
    win = gru_W_ih[2 * H:].T
    whr = gru_W_hh[:H].T
    whz = gru_W_hh[H:2 * H].T
    whn = gru_W_hh[2 * H:].T
    br = (gru_b_ih[:H] + gru_b_hh[:H]).reshape(1, H)
    bz = (gru_b_ih[H:2 * H] + gru_b_hh[H:2 * H]).reshape(1, H)
    bin_ = gru_b_ih[2 * H:].reshape(1, H)
    bhn = gru_b_hh[2 * H:].reshape(1, H)

    src = edge_index[0]
    dst = edge_index[1]
    pad = E_PAD - E
    src_p = jnp.concatenate([src, jnp.zeros((pad,), jnp.int32)]).reshape(EROWS, SUB)
    dst_conv = jnp.concatenate([dst, jnp.full((pad,), N, jnp.int32)]).reshape(EROWS, SUB)
    dst_mlp = jnp.concatenate([dst, jnp.zeros((pad,), jnp.int32)]).reshape(EROWS, SUB)
    ones_hbm = jnp.ones((TPT,), f32)

    deg = _deg_sc(dst_conv, ones_hbm)
    dinv, g1 = _gru_tc(x2, deg.reshape(N, 1), wir, wiz, win, whr, whz, whn,
                       br, bz, bin_, bhn, conv1_W)
    s1 = _conv_sc(g1, src_p, dst_conv)
    g2 = _mid_tc(s1, dinv, conv1_b.reshape(1, H), conv2_W)
    s2 = _conv_sc(g2, src_p, dst_conv)
    a, cc = _post_tc(s2, dinv, conv2_b.reshape(1, H),
                     mlp_W1[:H], mlp_b1.reshape(1, H), mlp_W1[H:])
    u = _mlp_sc(a, cc, src_p, dst_mlp)
    w24 = jax.scipy.linalg.block_diag(mlp_W2, mlp_W2, mlp_W2, mlp_W2)
    b24 = jnp.broadcast_to(mlp_b2.reshape(1, 1), (1, 4))
    out4 = _final_tc(u.reshape(E_PAD // 4, 4 * H), w24, b24)
    return out4.reshape(E_PAD)[:E]


# speedup vs baseline: 2.1205x; 1.1121x over previous
import functools

import jax
import jax.numpy as jnp
from jax import lax
from jax.experimental import pallas as pl
from jax.experimental.pallas import tpu as pltpu
from jax.experimental.pallas import tpu_sc as plsc

N = 100000
E = 1600000
SEQ = 8
IN = 16
H = 32

NC = 2
NS = 16
NHALF = N // NC
ACC_ROWS = 50176
TRASH = 50048
TPT = 3128
TPT_LO = 3080

SUB = 128
E_PAD = 1605632
EROWS = E_PAD // SUB

CR_CONV = 4
CPT_CONV = 196
CR_DEG = 16
CPT_DEG = 49
CR_MLP = 8
CPT_MLP = 49

BN = 2000
BE = 8192


def _sc_mesh():
    return plsc.VectorSubcoreMesh(core_axis_name="c", subcore_axis_name="s")


def _localize(idx_ref, b, nrows, coff):
    trash = TRASH + lax.iota(jnp.int32, 16)

    def body(j, _):
        r = j >> 3
        q = (j & 7) * 16
        d = idx_ref[b, r, pl.ds(q, 16)]
        dl = d - coff
        ok = (dl >= 0) & (dl < NHALF)
        idx_ref[b, r, pl.ds(q, 16)] = jnp.where(ok, dl, trash + (j & 7) * 16)
        return 0

    lax.fori_loop(0, nrows * 8, body, 0)


@functools.partial(
    pl.kernel,
    out_type=jax.ShapeDtypeStruct((N,), jnp.float32),
    mesh=_sc_mesh(),
    compiler_params=pltpu.CompilerParams(use_tc_tiling_on_sc=False),
    scratch_types=[
        pltpu.VMEM((2, CR_DEG, SUB), jnp.int32),
        pltpu.VMEM((TPT,), jnp.float32),
        pltpu.VMEM_SHARED((ACC_ROWS,), jnp.float32),
        pltpu.SemaphoreType.DMA,
        pltpu.SemaphoreType.DMA,
    ],
)
def _deg_sc(dst2, ones_hbm, deg_out, didx, vstage, acc, ssem, isem):
    c = lax.axis_index("c")
    s = lax.axis_index("s")
    coff = c * NHALF
    pltpu.sync_copy(ones_hbm.at[pl.ds(0, TPT)], vstage)
    pltpu.sync_copy(vstage, acc.at[pl.ds(s * TPT, TPT)])
    plsc.subcore_barrier()
    ones_v = vstage.at[pl.ds(0, SUB)]

    def _ifetch(i, b):
        rowbase = (s * CPT_DEG + i) * CR_DEG
        pltpu.async_copy(dst2.at[pl.ds(rowbase, CR_DEG)], didx.at[b], isem)

    def _iwait(i, b):
        rowbase = (s * CPT_DEG + i) * CR_DEG
        pltpu.make_async_copy(dst2.at[pl.ds(rowbase, CR_DEG)],
                              didx.at[b], isem).wait()

    _ifetch(0, 0)

    def chunk(i, _):
        b = i & 1

        @pl.when(i > 0)
        def _():
            for k in range(CR_DEG):
                pltpu.make_async_copy(ones_v, acc.at[didx.at[1 - b, k]],
                                      ssem).wait()

        _ifetch(jnp.minimum(i + 1, CPT_DEG - 1), 1 - b)
        _iwait(i, b)
        _localize(didx, b, CR_DEG, coff)
        for k in range(CR_DEG):
            pltpu.async_copy(ones_v, acc.at[didx.at[b, k]], ssem, add=True)
        return 0

    lax.fori_loop(0, CPT_DEG, chunk, 0)
    bl = (CPT_DEG - 1) & 1
    for k in range(CR_DEG):
        pltpu.make_async_copy(ones_v, acc.at[didx.at[bl, k]], ssem).wait()
    _iwait(CPT_DEG - 1, 1 - bl)
    plsc.subcore_barrier()
    base = c * NHALF + s * TPT
    pltpu.sync_copy(acc.at[pl.ds(s * TPT, TPT_LO)], vstage.at[pl.ds(0, TPT_LO)])
    pltpu.sync_copy(vstage.at[pl.ds(0, TPT_LO)], deg_out.at[pl.ds(base, TPT_LO)])

    @pl.when(s < NS - 1)
    def _():
        pltpu.sync_copy(acc.at[pl.ds(s * TPT + TPT_LO, TPT - TPT_LO)],
                        vstage.at[pl.ds(0, TPT - TPT_LO)])
        pltpu.sync_copy(vstage.at[pl.ds(0, TPT - TPT_LO)],
                        deg_out.at[pl.ds(base + TPT_LO, TPT - TPT_LO)])


@functools.partial(
    pl.kernel,
    out_type=jax.ShapeDtypeStruct((N, H), jnp.float32),
    mesh=_sc_mesh(),
    compiler_params=pltpu.CompilerParams(use_tc_tiling_on_sc=False),
    scratch_types=[
        pltpu.VMEM((2, CR_CONV, SUB), jnp.int32),
        pltpu.VMEM((2, CR_CONV, SUB), jnp.int32),
        pltpu.VMEM((CR_CONV * SUB, H), jnp.float32),
        pltpu.VMEM_SHARED((ACC_ROWS, H), jnp.float32),
        pltpu.SemaphoreType.DMA,
        pltpu.SemaphoreType.DMA,
        pltpu.SemaphoreType.DMA,
    ],
)
def _conv_sc(g, src2, dst2, s_out, sidx, didx, rows, acc, gsem, ssem, isem):
    c = lax.axis_index("c")
    s = lax.axis_index("s")
    coff = c * NHALF

    def _stage(src_ref, src_base, dst_ref, dst_base):
        piece0 = CR_CONV * SUB
        off = 0
        for piece in (piece0,) * 6 + (TPT_LO - 6 * piece0,):
            pltpu.sync_copy(src_ref.at[pl.ds(src_base + off, piece)],
                            rows.at[pl.ds(0, piece)])
            pltpu.sync_copy(rows.at[pl.ds(0, piece)],
                            dst_ref.at[pl.ds(dst_base + off, piece)])
            off += piece

        @pl.when(s < NS - 1)
        def _():
            pltpu.sync_copy(src_ref.at[pl.ds(src_base + TPT_LO, TPT - TPT_LO)],
                            rows.at[pl.ds(0, TPT - TPT_LO)])
            pltpu.sync_copy(rows.at[pl.ds(0, TPT - TPT_LO)],
                            dst_ref.at[pl.ds(dst_base + TPT_LO, TPT - TPT_LO)])

    _stage(g, coff + s * TPT, acc, s * TPT)
    plsc.subcore_barrier()

    def _ifetch(i, b):
        rowbase = (s * CPT_CONV + i) * CR_CONV
        pltpu.async_copy(src2.at[pl.ds(rowbase, CR_CONV)], sidx.at[b], isem)
        pltpu.async_copy(dst2.at[pl.ds(rowbase, CR_CONV)], didx.at[b], isem)

    def _iwait(i, b):
        rowbase = (s * CPT_CONV + i) * CR_CONV
        pltpu.make_async_copy(src2.at[pl.ds(rowbase, CR_CONV)],
                              sidx.at[b], isem).wait()
        pltpu.make_async_copy(dst2.at[pl.ds(rowbase, CR_CONV)],
                              didx.at[b], isem).wait()

    _ifetch(0, 0)

    def chunk(i, _):
        b = i & 1

        @pl.when(i > 0)
        def _():
            for k in range(CR_CONV):
                pltpu.make_async_copy(rows.at[pl.ds(k * SUB, SUB)],
                                      acc.at[didx.at[1 - b, k]], ssem).wait()

        _ifetch(jnp.minimum(i + 1, CPT_CONV - 1), 1 - b)
        _iwait(i, b)
        _localize(didx, b, CR_CONV, coff)
        pltpu.async_copy(g.at[sidx.at[b, 0]], rows.at[pl.ds(0, SUB)], gsem)
        for k in range(CR_CONV):
            if k + 1 < CR_CONV:
                pltpu.async_copy(g.at[sidx.at[b, k + 1]],
                                 rows.at[pl.ds((k + 1) * SUB, SUB)], gsem)
            pltpu.make_async_copy(g.at[sidx.at[b, k]],
                                  rows.at[pl.ds(k * SUB, SUB)], gsem).wait()
            pltpu.async_copy(rows.at[pl.ds(k * SUB, SUB)],
                             acc.at[didx.at[b, k]], ssem, add=True)
        return 0

    lax.fori_loop(0, CPT_CONV, chunk, 0)
    bl = (CPT_CONV - 1) & 1
    for k in range(CR_CONV):
        pltpu.make_async_copy(rows.at[pl.ds(k * SUB, SUB)],
                              acc.at[didx.at[bl, k]], ssem).wait()
    _iwait(CPT_CONV - 1, 1 - bl)
    plsc.subcore_barrier()
    _stage(acc, s * TPT, s_out, coff + s * TPT)


CMLP = CR_MLP * SUB


@functools.partial(
    pl.kernel,
    out_type=jax.ShapeDtypeStruct((E_PAD, H), jnp.float32),
    mesh=_sc_mesh(),
    compiler_params=pltpu.CompilerParams(use_tc_tiling_on_sc=False),
    scratch_types=[
        pltpu.VMEM((2, CR_MLP, SUB), jnp.int32),
        pltpu.VMEM((2, CR_MLP, SUB), jnp.int32),
        pltpu.VMEM((2, CMLP, H), jnp.float32),
        pltpu.SemaphoreType.DMA,
        pltpu.SemaphoreType.DMA,
        pltpu.SemaphoreType.DMA,
        pltpu.SemaphoreType.DMA,
    ],
)
def _mlp_sc(a, cc, src2, dst2, u_out, sidx, didx, rows, asem, csem, osem, isem):
    c = lax.axis_index("c")
    s = lax.axis_index("s")
    wid = c * NS + s

    def _ifetch(i, b):
        rowbase = (wid * CPT_MLP + i) * CR_MLP
        pltpu.async_copy(src2.at[pl.ds(rowbase, CR_MLP)], sidx.at[b], isem)
        pltpu.async_copy(dst2.at[pl.ds(rowbase, CR_MLP)], didx.at[b], isem)

    def _iwait(i, b):
        rowbase = (wid * CPT_MLP + i) * CR_MLP
        pltpu.make_async_copy(src2.at[pl.ds(rowbase, CR_MLP)],
                              sidx.at[b], isem).wait()
        pltpu.make_async_copy(dst2.at[pl.ds(rowbase, CR_MLP)],
                              didx.at[b], isem).wait()

    def _ocopy(i):
        base = (wid * CPT_MLP + i) * CMLP
        return (rows.at[i & 1], u_out.at[pl.ds(base, CMLP)], osem)

    _ifetch(0, 0)

    def chunk(i, _):
        b = i & 1

        @pl.when(i > 1)
        def _():
            so, do, _sem = _ocopy(i - 2)
            pltpu.make_async_copy(so, do, _sem).wait()

        _ifetch(jnp.minimum(i + 1, CPT_MLP - 1), 1 - b)
        _iwait(i, b)
        for k in range(CR_MLP):
            pltpu.async_copy(a.at[sidx.at[b, k]],
                             rows.at[b, pl.ds(k * SUB, SUB)], asem)
        for k in range(CR_MLP):
            pltpu.make_async_copy(a.at[sidx.at[b, k]],
                                  rows.at[b, pl.ds(k * SUB, SUB)], asem).wait()
            pltpu.async_copy(cc.at[didx.at[b, k]],
                             rows.at[b, pl.ds(k * SUB, SUB)], csem, add=True)
        for k in range(CR_MLP):
            pltpu.make_async_copy(cc.at[didx.at[b, k]],
                                  rows.at[b, pl.ds(k * SUB, SUB)], csem).wait()
        so, do, _sem = _ocopy(i)
        pltpu.async_copy(so, do, _sem)
        return 0

    lax.fori_loop(0, CPT_MLP, chunk, 0)
    so, do, _sem = _ocopy(CPT_MLP - 1)
    pltpu.make_async_copy(so, do, _sem).wait()
    so2, do2, _sem2 = _ocopy(CPT_MLP - 2)
    pltpu.make_async_copy(so2, do2, _sem2).wait()
    _iwait(CPT_MLP - 1, CPT_MLP & 1)


BNT = 2048


def _gru_body(x_ref, deg_ref, wih, whh, bi, bh, w1t, dinv_ref, g_ref):
    f32 = jnp.float32
    dn = (((1,), (1,)), ((), ()))
    h = jnp.zeros((H, BNT), f32)
    bi_r = bi[pl.ds(0, H)]
    bi_z = bi[pl.ds(H, H)]
    bi_n = bi[pl.ds(2 * H, H)]
    bh_r = bh[pl.ds(0, H)]
    bh_z = bh[pl.ds(H, H)]
    bh_n = bh[pl.ds(2 * H, H)]
    for t in range(SEQ):
        xt = x_ref[:, pl.ds(t * IN, IN)]
        gi = lax.dot_general(wih[...], xt, dn, preferred_element_type=f32)
        gh = jnp.dot(whh[...], h, preferred_element_type=f32)
        r = jax.nn.sigmoid(gi[0:H] + gh[0:H] + bi_r + bh_r)
        z = jax.nn.sigmoid(gi[H:2 * H] + gh[H:2 * H] + bi_z + bh_z)
        n = jnp.tanh(gi[2 * H:] + bi_n + r * (gh[2 * H:] + bh_n))
        h = (1.0 - z) * n + z * h
    dinv = lax.rsqrt(deg_ref[...])
    dinv_ref[...] = dinv
    g_ref[...] = jnp.dot(w1t[...], h, preferred_element_type=f32) * dinv


def _gru_tc(x2, degT, wih, whh, bi, bh, w1t):
    grid = (N + BNT - 1) // BNT
    return pl.pallas_call(
        _gru_body,
        grid=(grid,),
        in_specs=[pl.BlockSpec((BNT, SEQ * IN), lambda i: (i, 0)),
                  pl.BlockSpec((1, BNT), lambda i: (0, i)),
                  pl.BlockSpec((3 * H, IN), lambda i: (0, 0)),
                  pl.BlockSpec((3 * H, H), lambda i: (0, 0)),
                  pl.BlockSpec((3 * H, 1), lambda i: (0, 0)),
                  pl.BlockSpec((3 * H, 1), lambda i: (0, 0)),
                  pl.BlockSpec((H, H), lambda i: (0, 0))],
        out_specs=[pl.BlockSpec((1, BNT), lambda i: (0, i)),
                   pl.BlockSpec((H, BNT), lambda i: (0, i))],
        out_shape=[jax.ShapeDtypeStruct((1, N), jnp.float32),
                   jax.ShapeDtypeStruct((H, N), jnp.float32)],
    )(x2, degT, wih, whh, bi, bh, w1t)


def _mid_body(s_ref, dinv_ref, b_ref, w_ref, g_ref):
    h1 = jax.nn.relu(dinv_ref[...] * s_ref[...] + b_ref[...])
    g_ref[...] = jnp.dot(h1, w_ref[...],
                         preferred_element_type=jnp.float32) * dinv_ref[...]


def _mid_tc(s1, dinv, b, w):
    grid = N // BN
    return pl.pallas_call(
        _mid_body,
        grid=(grid,),
        in_specs=[pl.BlockSpec((BN, H), lambda i: (i, 0)),
                  pl.BlockSpec((BN, 1), lambda i: (i, 0)),
                  pl.BlockSpec((1, H), lambda i: (0, 0)),
                  pl.BlockSpec((H, H), lambda i: (0, 0))],
        out_specs=pl.BlockSpec((BN, H), lambda i: (i, 0)),
        out_shape=jax.ShapeDtypeStruct((N, H), jnp.float32),
    )(s1, dinv, b, w)


def _post_body(s_ref, dinv_ref, b_ref, wa_ref, ba_ref, wc_ref, a_ref, c_ref):
    h2 = jax.nn.relu(dinv_ref[...] * s_ref[...] + b_ref[...])
    a_ref[...] = jnp.dot(h2, wa_ref[...],
                         preferred_element_type=jnp.float32) + ba_ref[...]
    c_ref[...] = jnp.dot(h2, wc_ref[...], preferred_element_type=jnp.float32)


def _post_tc(s2, dinv, b, wa, ba, wc):
    grid = N // BN
    return pl.pallas_call(
        _post_body,
        grid=(grid,),
        in_specs=[pl.BlockSpec((BN, H), lambda i: (i, 0)),
                  pl.BlockSpec((BN, 1), lambda i: (i, 0)),
                  pl.BlockSpec((1, H), lambda i: (0, 0)),
                  pl.BlockSpec((H, H), lambda i: (0, 0)),
                  pl.BlockSpec((1, H), lambda i: (0, 0)),
                  pl.BlockSpec((H, H), lambda i: (0, 0))],
        out_specs=[pl.BlockSpec((BN, H), lambda i: (i, 0)),
                   pl.BlockSpec((BN, H), lambda i: (i, 0))],
        out_shape=[jax.ShapeDtypeStruct((N, H), jnp.float32),
                   jax.ShapeDtypeStruct((N, H), jnp.float32)],
    )(s2, dinv, b, wa, ba, wc)


def _final_body(u_ref, w_ref, b_ref, out_ref):
    out_ref[...] = (jnp.dot(jax.nn.relu(u_ref[...]), w_ref[...],
                            preferred_element_type=jnp.float32) + b_ref[...])


def _final_tc(u4, w24, b24):
    grid = E_PAD // 4 // 2048
    return pl.pallas_call(
        _final_body,
        grid=(grid,),
        in_specs=[pl.BlockSpec((2048, 4 * H), lambda i: (i, 0)),
                  pl.BlockSpec((4 * H, 4), lambda i: (0, 0)),
                  pl.BlockSpec((1, 4), lambda i: (0, 0))],
        out_specs=pl.BlockSpec((2048, 4), lambda i: (i, 0)),
        out_shape=jax.ShapeDtypeStruct((E_PAD // 4, 4), jnp.float32),
    )(u4, w24, b24)


def kernel(x, edge_index, gru_W_ih, gru_W_hh, gru_b_ih, gru_b_hh,
           conv1_W, conv1_b, conv2_W, conv2_b,
           mlp_W1, mlp_b1, mlp_W2, mlp_b2):
    f32 = jnp.float32
    x2 = x.reshape(N, SEQ * IN)


    src = edge_index[0]
    dst = edge_index[1]
    pad = E_PAD - E
    src_p = jnp.concatenate([src, jnp.zeros((pad,), jnp.int32)]).reshape(EROWS, SUB)
    dst_conv = jnp.concatenate([dst, jnp.full((pad,), N, jnp.int32)]).reshape(EROWS, SUB)
    dst_mlp = jnp.concatenate([dst, jnp.zeros((pad,), jnp.int32)]).reshape(EROWS, SUB)
    ones_hbm = jnp.ones((TPT,), f32)

    deg = _deg_sc(dst_conv, ones_hbm)
    dinvT, g1T = _gru_tc(x2, deg.reshape(1, N), gru_W_ih, gru_W_hh,
                         gru_b_ih.reshape(3 * H, 1), gru_b_hh.reshape(3 * H, 1),
                         conv1_W.T)
    dinv = dinvT.reshape(N, 1)
    g1 = g1T.T
    s1 = _conv_sc(g1, src_p, dst_conv)
    g2 = _mid_tc(s1, dinv, conv1_b.reshape(1, H), conv2_W)
    s2 = _conv_sc(g2, src_p, dst_conv)
    a, cc = _post_tc(s2, dinv, conv2_b.reshape(1, H),
                     mlp_W1[:H], mlp_b1.reshape(1, H), mlp_W1[H:])
    u = _mlp_sc(a, cc, src_p, dst_mlp)
    w24 = jax.scipy.linalg.block_diag(mlp_W2, mlp_W2, mlp_W2, mlp_W2)
    b24 = jnp.broadcast_to(mlp_b2.reshape(1, 1), (1, 4))
    out4 = _final_tc(u.reshape(E_PAD // 4, 4 * H), w24, b24)
    return out4.reshape(E_PAD)[:E]
